# baseline (speedup 1.0000x reference)
import functools

import jax
import jax.numpy as jnp
from jax import lax
from jax.experimental import pallas as pl
from jax.experimental.pallas import tpu as pltpu
from jax.experimental.pallas import tpu_sc as plsc

N_NODES = 10000
N_EDGES = 320000
D_FEAT = 128

NC = 2
NS = 16
NW = NC * NS

CHUNK = 64
N_CHUNKS = N_EDGES // CHUNK
N_ACC = 10240
ROWS_PER_TILE = N_ACC // NS
ZCOPIES = ROWS_PER_TILE // CHUNK
CG = 40
NBUF = 4

NBIG = 17
BIGN = 160
SMALLN = 152
SMALL_BASE = NBIG * BIGN

_mesh = plsc.VectorSubcoreMesh(core_axis_name="c", subcore_axis_name="s")


def _chunk_base(wid):
    return jnp.where(wid < NBIG, wid * BIGN,
                     SMALL_BASE + (wid - NBIG) * SMALLN)


@functools.partial(
    pl.kernel,
    out_type=(
        jax.ShapeDtypeStruct((NC, N_ACC), jnp.float32),
        jax.ShapeDtypeStruct((NC, N_ACC), jnp.float32),
    ),
    mesh=_mesh,
    scratch_types=[
        pltpu.VMEM((BIGN, CHUNK), jnp.int32),
        pltpu.VMEM((BIGN, CHUNK), jnp.int32),
        pltpu.VMEM((CHUNK,), jnp.float32),
        pltpu.VMEM((ROWS_PER_TILE,), jnp.float32),
        pltpu.VMEM_SHARED((N_ACC,), jnp.float32),
        pltpu.VMEM_SHARED((N_ACC,), jnp.float32),
    ],
)
def _hist_kernel(src_hbm, dst_hbm, hs_out, hd_out,
                 srcv, dstv, ones_v, zv, hs_sp, hd_sp):
    cid = lax.axis_index("c")
    sid = lax.axis_index("s")
    wid = sid * NC + cid
    base = _chunk_base(wid)

    @pl.when(wid < NBIG)
    def _():
        pltpu.sync_copy(src_hbm.at[pl.ds(base, BIGN)], srcv)
        pltpu.sync_copy(dst_hbm.at[pl.ds(base, BIGN)], dstv)

    @pl.when(wid >= NBIG)
    def _():
        pltpu.sync_copy(src_hbm.at[pl.ds(base, SMALLN)],
                        srcv.at[pl.ds(0, SMALLN)])
        pltpu.sync_copy(dst_hbm.at[pl.ds(base, SMALLN)],
                        dstv.at[pl.ds(0, SMALLN)])

    cnt = jnp.where(wid < NBIG, BIGN, SMALLN)

    @pl.loop(0, CHUNK // 16)
    def _(k):
        ones_v[pl.ds(k * 16, 16)] = jnp.ones((16,), jnp.float32)

    @pl.loop(0, ROWS_PER_TILE // 16)
    def _(k):
        zv[pl.ds(k * 16, 16)] = jnp.zeros((16,), jnp.float32)

    zbase = sid * ROWS_PER_TILE
    pltpu.sync_copy(zv, hs_sp.at[pl.ds(zbase, ROWS_PER_TILE)])
    pltpu.sync_copy(zv, hd_sp.at[pl.ds(zbase, ROWS_PER_TILE)])
    plsc.subcore_barrier()

    @pl.loop(0, cnt)
    def _(j):
        pltpu.sync_copy(ones_v, hs_sp.at[srcv.at[j]], add=True)
        pltpu.sync_copy(ones_v, hd_sp.at[dstv.at[j]], add=True)

    plsc.subcore_barrier()
    pltpu.sync_copy(hs_sp.at[pl.ds(zbase, ROWS_PER_TILE)],
                    hs_out.at[cid].at[pl.ds(zbase, ROWS_PER_TILE)])
    pltpu.sync_copy(hd_sp.at[pl.ds(zbase, ROWS_PER_TILE)],
                    hd_out.at[cid].at[pl.ds(zbase, ROWS_PER_TILE)])


@functools.partial(
    pl.kernel,
    out_type=jax.ShapeDtypeStruct((NC, N_ACC, D_FEAT), jnp.float32),
    mesh=_mesh,
    scratch_types=[
        pltpu.VMEM((CG, CHUNK), jnp.int32),
        pltpu.VMEM((CG, CHUNK), jnp.int32),
        [pltpu.VMEM((CHUNK, D_FEAT), jnp.float32)] * NBUF,
        pltpu.VMEM_SHARED((N_ACC, D_FEAT), jnp.float32),
        [pltpu.SemaphoreType.DMA] * NBUF,
    ],
)
def _agg_kernel(h_hbm, src_hbm, dst_hbm, out,
                srcv, dstv, rows, acc_sp, sems):
    cid = lax.axis_index("c")
    sid = lax.axis_index("s")
    wid = sid * NC + cid
    base = _chunk_base(wid)

    @pl.loop(0, CHUNK)
    def _(r):
        for c in range(D_FEAT // 16):
            rows[0][r, pl.ds(c * 16, 16)] = jnp.zeros((16,), jnp.float32)

    for k in range(ZCOPIES):
        off = (sid * ZCOPIES + k) * CHUNK
        pltpu.sync_copy(rows[0], acc_sp.at[pl.ds(off, CHUNK)])
    plsc.subcore_barrier()

    def process(base_chunk, n):
        pltpu.sync_copy(src_hbm.at[pl.ds(base_chunk, n)], srcv.at[pl.ds(0, n)])
        pltpu.sync_copy(dst_hbm.at[pl.ds(base_chunk, n)], dstv.at[pl.ds(0, n)])
        for k in range(NBUF - 1):
            pltpu.async_copy(h_hbm.at[srcv.at[k]], rows[k], sems[k])

        @pl.loop(0, n // NBUF)
        def _(i):
            j0 = NBUF * i
            for k in range(NBUF):
                j = j0 + k
                kn = (k + NBUF - 1) % NBUF

                @pl.when(j + NBUF - 1 < n)
                def _():
                    pltpu.async_copy(h_hbm.at[srcv.at[j + NBUF - 1]],
                                     rows[kn], sems[kn])

                pltpu.make_async_copy(h_hbm.at[srcv.at[j]],
                                      rows[k], sems[k]).wait()
                pltpu.sync_copy(rows[k], acc_sp.at[dstv.at[j]], add=True)

    @pl.when(wid < NBIG)
    def _():
        for g in range(BIGN // CG):
            process(base + g * CG, CG)

    @pl.when(wid >= NBIG)
    def _():
        for g in range(SMALLN // CG):
            process(base + g * CG, CG)
        process(base + (SMALLN // CG) * CG, SMALLN % CG)

    plsc.subcore_barrier()
    for k in range(ZCOPIES):
        off = (sid * ZCOPIES + k) * CHUNK
        pltpu.sync_copy(acc_sp.at[pl.ds(off, CHUNK)],
                        out.at[cid].at[pl.ds(off, CHUNK)])


def _scale_body(feat_ref, hs_ref, hd_ref, h_ref, nd_ref):
    deg_s = hs_ref[0, :N_NODES] + hs_ref[1, :N_NODES]
    deg_d = hd_ref[0, :N_NODES] + hd_ref[1, :N_NODES]
    norm_s = lax.rsqrt(jnp.maximum(deg_s, 1.0))
    h_ref[...] = feat_ref[...] * norm_s
    nd_ref[...] = lax.rsqrt(jnp.maximum(deg_d, 1.0))


def _final_body(p_ref, nd_ref, o_ref):
    o_ref[...] = (p_ref[0, :N_NODES] + p_ref[1, :N_NODES]) * nd_ref[...]


def kernel(feat, edge_index):
    er = edge_index.reshape(2, N_CHUNKS, CHUNK)
    src_r = er[0]
    dst_r = er[1]

    hs, hd = _hist_kernel(src_r, dst_r)

    h, norm_dst = pl.pallas_call(
        _scale_body,
        out_shape=(
            jax.ShapeDtypeStruct((N_NODES, D_FEAT), jnp.float32),
            jax.ShapeDtypeStruct((N_NODES, 1), jnp.float32),
        ),
    )(feat, hs[:, :, None], hd[:, :, None])

    partials = _agg_kernel(h, src_r, dst_r)

    out = pl.pallas_call(
        _final_body,
        out_shape=jax.ShapeDtypeStruct((N_NODES, D_FEAT), jnp.float32),
    )(partials, norm_dst)

    return out


# speedup vs baseline: 1.1366x; 1.1366x over previous
import functools

import jax
import jax.numpy as jnp
from jax import lax
from jax.experimental import pallas as pl
from jax.experimental.pallas import tpu as pltpu
from jax.experimental.pallas import tpu_sc as plsc

N_NODES = 10000
N_EDGES = 320000
D_FEAT = 128

NC = 2
NS = 16
NW = NC * NS

CHUNK = 128
N_CHUNKS = N_EDGES // CHUNK
N_ACC = 10240
ROWS_PER_TILE = N_ACC // NS
ZCOPIES = ROWS_PER_TILE // CHUNK
CG = 40
NBUF = 2

NBIG = 24
BIGN = 80
SMALLN = 72
SMALL_BASE = NBIG * BIGN
TAIL_BASE = SMALL_BASE + (NW - NBIG) * SMALLN
TAILN = N_CHUNKS - TAIL_BASE

_mesh = plsc.VectorSubcoreMesh(core_axis_name="c", subcore_axis_name="s")


def _chunk_base(wid):
    return jnp.where(wid < NBIG, wid * BIGN,
                     SMALL_BASE + (wid - NBIG) * SMALLN)


@functools.partial(
    pl.kernel,
    out_type=(
        jax.ShapeDtypeStruct((NC, N_ACC), jnp.float32),
        jax.ShapeDtypeStruct((NC, N_ACC), jnp.float32),
    ),
    mesh=_mesh,
    scratch_types=[
        pltpu.VMEM((BIGN, CHUNK), jnp.int32),
        pltpu.VMEM((BIGN, CHUNK), jnp.int32),
        pltpu.VMEM((CHUNK,), jnp.float32),
        pltpu.VMEM((ROWS_PER_TILE,), jnp.float32),
        pltpu.VMEM_SHARED((N_ACC,), jnp.float32),
        pltpu.VMEM_SHARED((N_ACC,), jnp.float32),
    ],
)
def _hist_kernel(e_hbm, hs_out, hd_out,
                 srcv, dstv, ones_v, zv, hs_sp, hd_sp):
    cid = lax.axis_index("c")
    sid = lax.axis_index("s")
    wid = sid * NC + cid
    base = _chunk_base(wid)
    src_hbm = e_hbm.at[0]
    dst_hbm = e_hbm.at[1]

    @pl.when(wid < NBIG)
    def _():
        pltpu.sync_copy(src_hbm.at[pl.ds(base, BIGN)], srcv)
        pltpu.sync_copy(dst_hbm.at[pl.ds(base, BIGN)], dstv)

    @pl.when(wid >= NBIG)
    def _():
        pltpu.sync_copy(src_hbm.at[pl.ds(base, SMALLN)],
                        srcv.at[pl.ds(0, SMALLN)])
        pltpu.sync_copy(dst_hbm.at[pl.ds(base, SMALLN)],
                        dstv.at[pl.ds(0, SMALLN)])

    @pl.when(wid == NW - 1)
    def _():
        pltpu.sync_copy(src_hbm.at[pl.ds(TAIL_BASE, TAILN)],
                        srcv.at[pl.ds(SMALLN, TAILN)])
        pltpu.sync_copy(dst_hbm.at[pl.ds(TAIL_BASE, TAILN)],
                        dstv.at[pl.ds(SMALLN, TAILN)])

    cnt = jnp.where(wid < NBIG, BIGN,
                    jnp.where(wid == NW - 1, SMALLN + TAILN, SMALLN))

    @pl.loop(0, CHUNK // 16)
    def _(k):
        ones_v[pl.ds(k * 16, 16)] = jnp.ones((16,), jnp.float32)

    @pl.loop(0, ROWS_PER_TILE // 16)
    def _(k):
        zv[pl.ds(k * 16, 16)] = jnp.zeros((16,), jnp.float32)

    zbase = sid * ROWS_PER_TILE
    pltpu.sync_copy(zv, hs_sp.at[pl.ds(zbase, ROWS_PER_TILE)])
    pltpu.sync_copy(zv, hd_sp.at[pl.ds(zbase, ROWS_PER_TILE)])
    plsc.subcore_barrier()

    @pl.loop(0, cnt)
    def _(j):
        pltpu.sync_copy(ones_v, hs_sp.at[srcv.at[j]], add=True)
        pltpu.sync_copy(ones_v, hd_sp.at[dstv.at[j]], add=True)

    plsc.subcore_barrier()
    pltpu.sync_copy(hs_sp.at[pl.ds(zbase, ROWS_PER_TILE)],
                    hs_out.at[cid].at[pl.ds(zbase, ROWS_PER_TILE)])
    pltpu.sync_copy(hd_sp.at[pl.ds(zbase, ROWS_PER_TILE)],
                    hd_out.at[cid].at[pl.ds(zbase, ROWS_PER_TILE)])


@functools.partial(
    pl.kernel,
    out_type=jax.ShapeDtypeStruct((NC, N_ACC, D_FEAT), jnp.float32),
    mesh=_mesh,
    scratch_types=[
        pltpu.VMEM((CG, CHUNK), jnp.int32),
        pltpu.VMEM((CG, CHUNK), jnp.int32),
        [pltpu.VMEM((CHUNK, D_FEAT), jnp.float32)] * NBUF,
        pltpu.VMEM_SHARED((N_ACC, D_FEAT), jnp.float32),
        [pltpu.SemaphoreType.DMA] * NBUF,
    ],
)
def _agg_kernel(h_hbm, e_hbm, out,
                srcv, dstv, rows, acc_sp, sems):
    cid = lax.axis_index("c")
    sid = lax.axis_index("s")
    wid = sid * NC + cid
    base = _chunk_base(wid)
    src_hbm = e_hbm.at[0]
    dst_hbm = e_hbm.at[1]

    @pl.loop(0, CHUNK)
    def _(r):
        for c in range(D_FEAT // 16):
            rows[0][r, pl.ds(c * 16, 16)] = jnp.zeros((16,), jnp.float32)

    for k in range(ZCOPIES):
        off = (sid * ZCOPIES + k) * CHUNK
        pltpu.sync_copy(rows[0], acc_sp.at[pl.ds(off, CHUNK)])
    plsc.subcore_barrier()

    def process(base_chunk, n):
        pltpu.sync_copy(src_hbm.at[pl.ds(base_chunk, n)], srcv.at[pl.ds(0, n)])
        pltpu.sync_copy(dst_hbm.at[pl.ds(base_chunk, n)], dstv.at[pl.ds(0, n)])
        for k in range(NBUF - 1):
            pltpu.async_copy(h_hbm.at[srcv.at[k]], rows[k], sems[k])

        @pl.loop(0, n // NBUF)
        def _(i):
            j0 = NBUF * i
            for k in range(NBUF):
                j = j0 + k
                kn = (k + NBUF - 1) % NBUF

                @pl.when(j + NBUF - 1 < n)
                def _():
                    pltpu.async_copy(h_hbm.at[srcv.at[j + NBUF - 1]],
                                     rows[kn], sems[kn])

                pltpu.make_async_copy(h_hbm.at[srcv.at[j]],
                                      rows[k], sems[k]).wait()
                pltpu.sync_copy(rows[k], acc_sp.at[dstv.at[j]], add=True)

    @pl.when(wid < NBIG)
    def _():
        for g in range(BIGN // CG):
            process(base + g * CG, CG)

    @pl.when(wid >= NBIG)
    def _():
        process(base, CG)
        process(base + CG, SMALLN - CG)

    @pl.when(wid == NW - 1)
    def _():
        process(TAIL_BASE, TAILN)

    plsc.subcore_barrier()
    for k in range(ZCOPIES):
        off = (sid * ZCOPIES + k) * CHUNK
        pltpu.sync_copy(acc_sp.at[pl.ds(off, CHUNK)],
                        out.at[cid].at[pl.ds(off, CHUNK)])


def _scale_body(feat_ref, hs_ref, hd_ref, h_ref, nd_ref):
    deg_s = hs_ref[0, :N_NODES] + hs_ref[1, :N_NODES]
    deg_d = hd_ref[0, :N_NODES] + hd_ref[1, :N_NODES]
    norm_s = lax.rsqrt(jnp.maximum(deg_s, 1.0))
    h_ref[...] = feat_ref[...] * norm_s
    nd_ref[...] = lax.rsqrt(jnp.maximum(deg_d, 1.0))


def _final_body(p_ref, nd_ref, o_ref):
    o_ref[...] = (p_ref[0, :N_NODES] + p_ref[1, :N_NODES]) * nd_ref[...]


def kernel(feat, edge_index):
    er = edge_index.reshape(2, N_CHUNKS, CHUNK)

    hs, hd = _hist_kernel(er)

    h, norm_dst = pl.pallas_call(
        _scale_body,
        out_shape=(
            jax.ShapeDtypeStruct((N_NODES, D_FEAT), jnp.float32),
            jax.ShapeDtypeStruct((N_NODES, 1), jnp.float32),
        ),
    )(feat, hs[:, :, None], hd[:, :, None])

    partials = _agg_kernel(h, er)

    out = pl.pallas_call(
        _final_body,
        out_shape=jax.ShapeDtypeStruct((N_NODES, D_FEAT), jnp.float32),
    )(partials, norm_dst)

    return out


# speedup vs baseline: 1.2794x; 1.1256x over previous
import functools

import jax
import jax.numpy as jnp
from jax import lax
from jax.experimental import pallas as pl
from jax.experimental.pallas import tpu as pltpu
from jax.experimental.pallas import tpu_sc as plsc

N_NODES = 10000
N_EDGES = 320000
D_FEAT = 128

NC = 2
NS = 16
NW = NC * NS

CHUNK = 128
N_CHUNKS = N_EDGES // CHUNK
N_ACC = 10240
ROWS_PER_TILE = N_ACC // NS
ZCOPIES = ROWS_PER_TILE // CHUNK
CG = 40
NBUF = 2

NBIG = 24
BIGN = 80
SMALLN = 72
SMALL_BASE = NBIG * BIGN
TAIL_BASE = SMALL_BASE + (NW - NBIG) * SMALLN
TAILN = N_CHUNKS - TAIL_BASE

_mesh = plsc.VectorSubcoreMesh(core_axis_name="c", subcore_axis_name="s")


def _chunk_base(wid):
    return jnp.where(wid < NBIG, wid * BIGN,
                     SMALL_BASE + (wid - NBIG) * SMALLN)


@functools.partial(
    pl.kernel,
    out_type=(
        jax.ShapeDtypeStruct((NC, N_ACC), jnp.float32),
        jax.ShapeDtypeStruct((NC, N_ACC), jnp.float32),
    ),
    mesh=_mesh,
    scratch_types=[
        pltpu.VMEM((BIGN, CHUNK), jnp.int32),
        pltpu.VMEM((BIGN, CHUNK), jnp.int32),
        pltpu.VMEM((CHUNK,), jnp.float32),
        pltpu.VMEM((ROWS_PER_TILE,), jnp.float32),
        pltpu.VMEM_SHARED((N_ACC,), jnp.float32),
        pltpu.VMEM_SHARED((N_ACC,), jnp.float32),
    ],
)
def _hist_kernel(e_hbm, hs_out, hd_out,
                 srcv, dstv, ones_v, zv, hs_sp, hd_sp):
    cid = lax.axis_index("c")
    sid = lax.axis_index("s")
    wid = sid * NC + cid
    base = _chunk_base(wid)
    src_hbm = e_hbm.at[0]
    dst_hbm = e_hbm.at[1]

    @pl.when(wid < NBIG)
    def _():
        pltpu.sync_copy(src_hbm.at[pl.ds(base, BIGN)], srcv)
        pltpu.sync_copy(dst_hbm.at[pl.ds(base, BIGN)], dstv)

    @pl.when(wid >= NBIG)
    def _():
        pltpu.sync_copy(src_hbm.at[pl.ds(base, SMALLN)],
                        srcv.at[pl.ds(0, SMALLN)])
        pltpu.sync_copy(dst_hbm.at[pl.ds(base, SMALLN)],
                        dstv.at[pl.ds(0, SMALLN)])

    @pl.when(wid == NW - 1)
    def _():
        pltpu.sync_copy(src_hbm.at[pl.ds(TAIL_BASE, TAILN)],
                        srcv.at[pl.ds(SMALLN, TAILN)])
        pltpu.sync_copy(dst_hbm.at[pl.ds(TAIL_BASE, TAILN)],
                        dstv.at[pl.ds(SMALLN, TAILN)])

    cnt = jnp.where(wid < NBIG, BIGN,
                    jnp.where(wid == NW - 1, SMALLN + TAILN, SMALLN))

    @pl.loop(0, CHUNK // 16)
    def _(k):
        ones_v[pl.ds(k * 16, 16)] = jnp.ones((16,), jnp.float32)

    @pl.loop(0, ROWS_PER_TILE // 16)
    def _(k):
        zv[pl.ds(k * 16, 16)] = jnp.zeros((16,), jnp.float32)

    zbase = sid * ROWS_PER_TILE
    pltpu.sync_copy(zv, hs_sp.at[pl.ds(zbase, ROWS_PER_TILE)])
    pltpu.sync_copy(zv, hd_sp.at[pl.ds(zbase, ROWS_PER_TILE)])
    plsc.subcore_barrier()

    @pl.loop(0, cnt)
    def _(j):
        pltpu.sync_copy(ones_v, hs_sp.at[srcv.at[j]], add=True)
        pltpu.sync_copy(ones_v, hd_sp.at[dstv.at[j]], add=True)

    plsc.subcore_barrier()
    pltpu.sync_copy(hs_sp.at[pl.ds(zbase, ROWS_PER_TILE)],
                    hs_out.at[cid].at[pl.ds(zbase, ROWS_PER_TILE)])
    pltpu.sync_copy(hd_sp.at[pl.ds(zbase, ROWS_PER_TILE)],
                    hd_out.at[cid].at[pl.ds(zbase, ROWS_PER_TILE)])


@functools.partial(
    pl.kernel,
    out_type=jax.ShapeDtypeStruct((NC, N_ACC, D_FEAT), jnp.float32),
    mesh=_mesh,
    scratch_types=[
        pltpu.VMEM((CG, CHUNK), jnp.int32),
        pltpu.VMEM((CG, CHUNK), jnp.int32),
        [pltpu.VMEM((CHUNK, D_FEAT), jnp.float32)] * NBUF,
        pltpu.VMEM_SHARED((N_ACC, D_FEAT), jnp.float32),
        [pltpu.SemaphoreType.DMA] * NBUF,
    ],
)
def _agg_kernel(h_hbm, e_hbm, out,
                srcv, dstv, rows, acc_sp, sems):
    cid = lax.axis_index("c")
    sid = lax.axis_index("s")
    wid = sid * NC + cid
    base = _chunk_base(wid)
    src_hbm = e_hbm.at[0]
    dst_hbm = e_hbm.at[1]

    @pl.loop(0, CHUNK)
    def _(r):
        for c in range(D_FEAT // 16):
            rows[0][r, pl.ds(c * 16, 16)] = jnp.zeros((16,), jnp.float32)

    for k in range(ZCOPIES):
        off = (sid * ZCOPIES + k) * CHUNK
        pltpu.sync_copy(rows[0], acc_sp.at[pl.ds(off, CHUNK)])
    plsc.subcore_barrier()

    def process(base_chunk, n):
        pltpu.sync_copy(src_hbm.at[pl.ds(base_chunk, n)], srcv.at[pl.ds(0, n)])
        pltpu.sync_copy(dst_hbm.at[pl.ds(base_chunk, n)], dstv.at[pl.ds(0, n)])
        for k in range(NBUF - 1):
            pltpu.async_copy(h_hbm.at[srcv.at[k]], rows[k], sems[k])

        @pl.loop(0, n // NBUF)
        def _(i):
            j0 = NBUF * i
            for k in range(NBUF):
                j = j0 + k
                kn = (k + NBUF - 1) % NBUF

                @pl.when(j + NBUF - 1 < n)
                def _():
                    pltpu.async_copy(h_hbm.at[srcv.at[j + NBUF - 1]],
                                     rows[kn], sems[kn])

                pltpu.make_async_copy(h_hbm.at[srcv.at[j]],
                                      rows[k], sems[k]).wait()
                pltpu.sync_copy(rows[k], acc_sp.at[dstv.at[j]], add=True)

    @pl.when(wid < NBIG)
    def _():
        for g in range(BIGN // CG):
            process(base + g * CG, CG)

    @pl.when(wid >= NBIG)
    def _():
        process(base, CG)
        process(base + CG, SMALLN - CG)

    @pl.when(wid == NW - 1)
    def _():
        process(TAIL_BASE, TAILN)

    plsc.subcore_barrier()
    for k in range(ZCOPIES):
        off = (sid * ZCOPIES + k) * CHUNK
        pltpu.sync_copy(acc_sp.at[pl.ds(off, CHUNK)],
                        out.at[cid].at[pl.ds(off, CHUNK)])


def _scale_body(feat_ref, hs_ref, hd_ref, h_ref, nd_ref):
    deg_s = hs_ref[0] + hs_ref[1]
    deg_d = hd_ref[0] + hd_ref[1]
    norm_s = lax.rsqrt(jnp.maximum(deg_s[:N_NODES], 1.0))
    h_ref[...] = feat_ref[...] * jnp.reshape(norm_s, (N_NODES, 1))
    nd_ref[...] = lax.rsqrt(jnp.maximum(deg_d[:N_NODES], 1.0))


def _final_body(p_ref, nd_ref, o_ref):
    nd = jnp.reshape(nd_ref[...], (N_NODES, 1))
    o_ref[...] = (p_ref[0, :N_NODES] + p_ref[1, :N_NODES]) * nd


def kernel(feat, edge_index):
    er = edge_index.reshape(2, N_CHUNKS, CHUNK)

    hs, hd = _hist_kernel(er)

    h, norm_dst = pl.pallas_call(
        _scale_body,
        out_shape=(
            jax.ShapeDtypeStruct((N_NODES, D_FEAT), jnp.float32),
            jax.ShapeDtypeStruct((N_NODES,), jnp.float32),
        ),
    )(feat, hs, hd)

    partials = _agg_kernel(h, er)

    out = pl.pallas_call(
        _final_body,
        out_shape=jax.ShapeDtypeStruct((N_NODES, D_FEAT), jnp.float32),
    )(partials, norm_dst)

    return out


# speedup vs baseline: 1.3254x; 1.0359x over previous
import functools

import jax
import jax.numpy as jnp
from jax import lax
from jax.experimental import pallas as pl
from jax.experimental.pallas import tpu as pltpu
from jax.experimental.pallas import tpu_sc as plsc

N_NODES = 10000
N_EDGES = 320000
D_FEAT = 128

NC = 2
NS = 16
NW = NC * NS

CHUNK = 128
N_CHUNKS = N_EDGES // CHUNK
N_ACC = 10240
ROWS_PER_TILE = N_ACC // NS
ZCOPIES = ROWS_PER_TILE // CHUNK
CG = 40
NBUF = 2

NBIG = 24
BIGN = 80
SMALLN = 72
SMALL_BASE = NBIG * BIGN
TAIL_BASE = SMALL_BASE + (NW - NBIG) * SMALLN
TAILN = N_CHUNKS - TAIL_BASE

_mesh = plsc.VectorSubcoreMesh(core_axis_name="c", subcore_axis_name="s")


def _chunk_base(wid):
    return jnp.where(wid < NBIG, wid * BIGN,
                     SMALL_BASE + (wid - NBIG) * SMALLN)


@functools.partial(
    pl.kernel,
    out_type=(
        jax.ShapeDtypeStruct((NC, N_ACC), jnp.float32),
        jax.ShapeDtypeStruct((NC, N_ACC), jnp.float32),
    ),
    mesh=_mesh,
    scratch_types=[
        pltpu.VMEM((BIGN, CHUNK), jnp.int32),
        pltpu.VMEM((BIGN, CHUNK), jnp.int32),
        pltpu.VMEM((CHUNK,), jnp.float32),
        pltpu.VMEM((ROWS_PER_TILE,), jnp.float32),
        pltpu.VMEM_SHARED((N_ACC,), jnp.float32),
        pltpu.VMEM_SHARED((N_ACC,), jnp.float32),
        pltpu.SemaphoreType.DMA,
        pltpu.SemaphoreType.DMA,
    ],
)
def _hist_kernel(e_hbm, hs_out, hd_out,
                 srcv, dstv, ones_v, zv, hs_sp, hd_sp, hsem0, hsem1):
    cid = lax.axis_index("c")
    sid = lax.axis_index("s")
    wid = sid * NC + cid
    base = _chunk_base(wid)
    src_hbm = e_hbm.at[0]
    dst_hbm = e_hbm.at[1]

    @pl.when(wid < NBIG)
    def _():
        pltpu.sync_copy(src_hbm.at[pl.ds(base, BIGN)], srcv)
        pltpu.sync_copy(dst_hbm.at[pl.ds(base, BIGN)], dstv)

    @pl.when(wid >= NBIG)
    def _():
        pltpu.sync_copy(src_hbm.at[pl.ds(base, SMALLN)],
                        srcv.at[pl.ds(0, SMALLN)])
        pltpu.sync_copy(dst_hbm.at[pl.ds(base, SMALLN)],
                        dstv.at[pl.ds(0, SMALLN)])

    @pl.when(wid == NW - 1)
    def _():
        pltpu.sync_copy(src_hbm.at[pl.ds(TAIL_BASE, TAILN)],
                        srcv.at[pl.ds(SMALLN, TAILN)])
        pltpu.sync_copy(dst_hbm.at[pl.ds(TAIL_BASE, TAILN)],
                        dstv.at[pl.ds(SMALLN, TAILN)])

    cnt = jnp.where(wid < NBIG, BIGN,
                    jnp.where(wid == NW - 1, SMALLN + TAILN, SMALLN))

    @pl.loop(0, CHUNK // 16)
    def _(k):
        ones_v[pl.ds(k * 16, 16)] = jnp.ones((16,), jnp.float32)

    @pl.loop(0, ROWS_PER_TILE // 16)
    def _(k):
        zv[pl.ds(k * 16, 16)] = jnp.zeros((16,), jnp.float32)

    zbase = sid * ROWS_PER_TILE
    pltpu.sync_copy(zv, hs_sp.at[pl.ds(zbase, ROWS_PER_TILE)])
    pltpu.sync_copy(zv, hd_sp.at[pl.ds(zbase, ROWS_PER_TILE)])
    plsc.subcore_barrier()

    @pl.loop(0, cnt)
    def _(j):
        d0 = pltpu.async_copy(ones_v, hs_sp.at[srcv.at[j]], hsem0, add=True)
        d1 = pltpu.async_copy(ones_v, hd_sp.at[dstv.at[j]], hsem1, add=True)
        d0.wait()
        d1.wait()

    plsc.subcore_barrier()
    pltpu.sync_copy(hs_sp.at[pl.ds(zbase, ROWS_PER_TILE)],
                    hs_out.at[cid].at[pl.ds(zbase, ROWS_PER_TILE)])
    pltpu.sync_copy(hd_sp.at[pl.ds(zbase, ROWS_PER_TILE)],
                    hd_out.at[cid].at[pl.ds(zbase, ROWS_PER_TILE)])


@functools.partial(
    pl.kernel,
    out_type=jax.ShapeDtypeStruct((NC, N_ACC, D_FEAT), jnp.float32),
    mesh=_mesh,
    scratch_types=[
        pltpu.VMEM((CG, CHUNK), jnp.int32),
        pltpu.VMEM((CG, CHUNK), jnp.int32),
        [pltpu.VMEM((CHUNK, D_FEAT), jnp.float32)] * NBUF,
        pltpu.VMEM_SHARED((N_ACC, D_FEAT), jnp.float32),
        [pltpu.SemaphoreType.DMA] * NBUF,
    ],
)
def _agg_kernel(h_hbm, e_hbm, out,
                srcv, dstv, rows, acc_sp, sems):
    cid = lax.axis_index("c")
    sid = lax.axis_index("s")
    wid = sid * NC + cid
    base = _chunk_base(wid)
    src_hbm = e_hbm.at[0]
    dst_hbm = e_hbm.at[1]

    @pl.loop(0, CHUNK)
    def _(r):
        for c in range(D_FEAT // 16):
            rows[0][r, pl.ds(c * 16, 16)] = jnp.zeros((16,), jnp.float32)

    for k in range(ZCOPIES):
        off = (sid * ZCOPIES + k) * CHUNK
        pltpu.sync_copy(rows[0], acc_sp.at[pl.ds(off, CHUNK)])
    plsc.subcore_barrier()

    def process(base_chunk, n):
        pltpu.sync_copy(src_hbm.at[pl.ds(base_chunk, n)], srcv.at[pl.ds(0, n)])
        pltpu.sync_copy(dst_hbm.at[pl.ds(base_chunk, n)], dstv.at[pl.ds(0, n)])
        for k in range(NBUF - 1):
            pltpu.async_copy(h_hbm.at[srcv.at[k]], rows[k], sems[k])

        @pl.loop(0, n // NBUF)
        def _(i):
            j0 = NBUF * i
            for k in range(NBUF):
                j = j0 + k
                kn = (k + NBUF - 1) % NBUF

                @pl.when(j + NBUF - 1 < n)
                def _():
                    pltpu.async_copy(h_hbm.at[srcv.at[j + NBUF - 1]],
                                     rows[kn], sems[kn])

                pltpu.make_async_copy(h_hbm.at[srcv.at[j]],
                                      rows[k], sems[k]).wait()
                pltpu.sync_copy(rows[k], acc_sp.at[dstv.at[j]], add=True)

    @pl.when(wid < NBIG)
    def _():
        for g in range(BIGN // CG):
            process(base + g * CG, CG)

    @pl.when(wid >= NBIG)
    def _():
        process(base, CG)
        process(base + CG, SMALLN - CG)

    @pl.when(wid == NW - 1)
    def _():
        process(TAIL_BASE, TAILN)

    plsc.subcore_barrier()
    for k in range(ZCOPIES):
        off = (sid * ZCOPIES + k) * CHUNK
        pltpu.sync_copy(acc_sp.at[pl.ds(off, CHUNK)],
                        out.at[cid].at[pl.ds(off, CHUNK)])


def _scale_body(feat_ref, hs_ref, hd_ref, h_ref, nd_ref):
    deg_s = hs_ref[0] + hs_ref[1]
    deg_d = hd_ref[0] + hd_ref[1]
    norm_s = lax.rsqrt(jnp.maximum(deg_s[:N_NODES], 1.0))
    h_ref[...] = feat_ref[...] * jnp.reshape(norm_s, (N_NODES, 1))
    nd_ref[...] = lax.rsqrt(jnp.maximum(deg_d[:N_NODES], 1.0))


def _final_body(p_ref, nd_ref, o_ref):
    nd = jnp.reshape(nd_ref[...], (N_NODES, 1))
    o_ref[...] = (p_ref[0, :N_NODES] + p_ref[1, :N_NODES]) * nd


def kernel(feat, edge_index):
    er = edge_index.reshape(2, N_CHUNKS, CHUNK)

    hs, hd = _hist_kernel(er)

    h, norm_dst = pl.pallas_call(
        _scale_body,
        out_shape=(
            jax.ShapeDtypeStruct((N_NODES, D_FEAT), jnp.float32),
            jax.ShapeDtypeStruct((N_NODES,), jnp.float32),
        ),
    )(feat, hs, hd)

    partials = _agg_kernel(h, er)

    out = pl.pallas_call(
        _final_body,
        out_shape=jax.ShapeDtypeStruct((N_NODES, D_FEAT), jnp.float32),
    )(partials, norm_dst)

    return out
